# baseline (speedup 1.0000x reference)
import functools

import jax
import jax.numpy as jnp
from jax import lax
from jax.experimental import pallas as pl
from jax.experimental.pallas import tpu as pltpu
from jax.experimental.pallas import tpu_sc as plsc

N = 16777216
NBINS = 256
_FINE = NBINS * 16


_MM_ROWS = 256
_MM_COLS = 8192
_MM_ACC = (8, 1024)


def _minmax_body(x_ref, mm_ref, amin_ref, amax_ref):
    i = pl.program_id(0)
    blk = x_ref[...].reshape(-1, *_MM_ACC)
    bmin = jnp.min(blk, axis=0)
    bmax = jnp.max(blk, axis=0)

    @pl.when(i == 0)
    def _init():
        amin_ref[...] = bmin
        amax_ref[...] = bmax

    @pl.when(i > 0)
    def _acc():
        amin_ref[...] = jnp.minimum(amin_ref[...], bmin)
        amax_ref[...] = jnp.maximum(amax_ref[...], bmax)

    @pl.when(i == pl.num_programs(0) - 1)
    def _fin():
        mm_ref[...] = jnp.concatenate([
            jnp.full((1, 16), jnp.min(amin_ref[...]), jnp.float32),
            jnp.full((1, 16), jnp.max(amax_ref[...]), jnp.float32),
        ])


def _minmax_tc(x2d, cover_rows):
    grid = cover_rows // _MM_ROWS
    return pl.pallas_call(
        _minmax_body,
        grid=(grid,),
        in_specs=[pl.BlockSpec((_MM_ROWS, _MM_COLS), lambda i: (i, 0))],
        out_specs=pl.BlockSpec((2, 16), lambda i: (0, 0)),
        out_shape=jax.ShapeDtypeStruct((2, 16), jnp.float32),
        scratch_shapes=[
            pltpu.VMEM(_MM_ACC, jnp.float32),
            pltpu.VMEM(_MM_ACC, jnp.float32),
        ],
    )(x2d)


_MMSC_CHUNK = 32768
_MMSC_UNROLL = 8


def _minmax_sc(x, lo_elem):
    nh = N - lo_elem
    pw = nh // _NW
    nchunk = pw // _MMSC_CHUNK
    mesh = plsc.VectorSubcoreMesh(core_axis_name="c", subcore_axis_name="s")

    @functools.partial(
        pl.kernel,
        mesh=mesh,
        out_type=(
            jax.ShapeDtypeStruct((_NW, 16), jnp.float32),
            jax.ShapeDtypeStruct((_NW, 16), jnp.float32),
        ),
        scratch_types=[
            pltpu.VMEM((2, _MMSC_CHUNK), jnp.float32),
            pltpu.VMEM((16,), jnp.float32),
            pltpu.VMEM((16,), jnp.float32),
            pltpu.SemaphoreType.DMA,
            pltpu.SemaphoreType.DMA,
        ],
        compiler_params=pltpu.CompilerParams(needs_layout_passes=False),
    )
    def mm_kernel(x_hbm, lo_hbm, hi_hbm, bufs_v, lo_v, hi_v, sem0, sem1):
        cid = lax.axis_index("c")
        sid = lax.axis_index("s")
        wid = sid * 2 + cid
        base = lo_elem + wid * pw
        sems = (sem0, sem1)

        def start(c):
            b = c & 1
            return pltpu.async_copy(
                x_hbm.at[pl.ds(base + c * _MMSC_CHUNK, _MMSC_CHUNK)],
                bufs_v.at[b], sems[b],
            )

        def make_vec_body(b):
            def vec_body(j, carry):
                off = j * (16 * _MMSC_UNROLL)
                mins = list(carry[:_MMSC_UNROLL])
                maxs = list(carry[_MMSC_UNROLL:])
                for k in range(_MMSC_UNROLL):
                    v = bufs_v[b, pl.ds(off + k * 16, 16)]
                    mins[k] = jnp.minimum(mins[k], v)
                    maxs[k] = jnp.maximum(maxs[k], v)
                return tuple(mins) + tuple(maxs)
            return vec_body

        pinf = jnp.full((16,), jnp.inf, dtype=jnp.float32)
        ninf = jnp.full((16,), -jnp.inf, dtype=jnp.float32)
        carry = (pinf,) * _MMSC_UNROLL + (ninf,) * _MMSC_UNROLL

        copies = [start(0)]
        for c in range(nchunk):
            if c + 1 < nchunk:
                copies.append(start(c + 1))
            copies[c].wait()
            carry = lax.fori_loop(
                0, _MMSC_CHUNK // (16 * _MMSC_UNROLL), make_vec_body(c & 1),
                carry)

        amin, amax = carry[0], carry[_MMSC_UNROLL]
        for k in range(1, _MMSC_UNROLL):
            amin = jnp.minimum(amin, carry[k])
            amax = jnp.maximum(amax, carry[_MMSC_UNROLL + k])
        lo_v[...] = amin
        hi_v[...] = amax
        pltpu.sync_copy(lo_v, lo_hbm.at[wid])
        pltpu.sync_copy(hi_v, hi_hbm.at[wid])

    return mm_kernel(x)


_NW = 32
_PW = N // _NW
_CHUNK = 32768
_NCHUNK = _PW // _CHUNK
_UNROLL = 32


def _sc_hist(x, tcmm, sc_los, sc_his):
    mesh = plsc.VectorSubcoreMesh(core_axis_name="c", subcore_axis_name="s")

    @functools.partial(
        pl.kernel,
        mesh=mesh,
        out_type=jax.ShapeDtypeStruct((_NW, NBINS), jnp.float32),
        scratch_types=[
            pltpu.VMEM((2, 16), jnp.float32),
            pltpu.VMEM((_NW, 16), jnp.float32),
            pltpu.VMEM((_NW, 16), jnp.float32),
            pltpu.VMEM((2, _CHUNK), jnp.float32),
            pltpu.VMEM((_FINE,), jnp.float32),
            pltpu.VMEM((NBINS,), jnp.float32),
            pltpu.SemaphoreType.DMA,
            pltpu.SemaphoreType.DMA,
        ],
        compiler_params=pltpu.CompilerParams(needs_layout_passes=False),
    )
    def hist_kernel(x_hbm, tcmm_hbm, los_hbm, his_hbm, out_hbm, tcmm_v,
                    los_v, his_v, bufs_v, fine_v, fold_v, sem0, sem1):
        cid = lax.axis_index("c")
        sid = lax.axis_index("s")
        wid = sid * 2 + cid
        base = wid * _PW
        sems = (sem0, sem1)

        pltpu.sync_copy(tcmm_hbm, tcmm_v)
        pltpu.sync_copy(los_hbm, los_v)
        pltpu.sync_copy(his_hbm, his_v)
        lo_v = tcmm_v[0]
        hi_v = tcmm_v[1]
        for w in range(_NW):
            lo_v = jnp.minimum(lo_v, los_v[w])
            hi_v = jnp.maximum(hi_v, his_v[w])
        lo_v = jnp.broadcast_to(jnp.min(lo_v), (16,))
        hi_v = jnp.broadcast_to(jnp.max(hi_v), (16,))
        scale16_v = jnp.full((16,), 4096.0 * (1.0 - 2.0**-18),
                             dtype=jnp.float32) / (hi_v - lo_v)
        ones = jnp.full((16,), 1.0, dtype=jnp.float32)
        zeros = jnp.zeros((16,), dtype=jnp.float32)

        def zero_body(j, _):
            fine_v[pl.ds(j * 16, 16)] = zeros
            return 0

        lax.fori_loop(0, _FINE // 16, zero_body, 0)

        def start(c):
            b = c & 1
            return pltpu.async_copy(
                x_hbm.at[pl.ds(base + c * _CHUNK, _CHUNK)],
                bufs_v.at[b], sems[b],
            )

        def make_vec_body(b):
            def vec_body(j, acc):
                off = j * (16 * _UNROLL)
                idxs = []
                for k in range(_UNROLL):
                    v = bufs_v[b, pl.ds(off + k * 16, 16)]
                    idxs.append(((v - lo_v) * scale16_v).astype(jnp.int32))
                for idx in idxs:
                    plsc.addupdate_scatter(fine_v, [idx], ones)
                return acc
            return vec_body

        acc = jnp.zeros((16,), dtype=jnp.int32)
        copies = [start(0)]
        for c in range(_NCHUNK):
            if c + 1 < _NCHUNK:
                copies.append(start(c + 1))
            copies[c].wait()
            acc = lax.fori_loop(0, _CHUNK // (16 * _UNROLL),
                                make_vec_body(c & 1), acc)

        lanes = lax.iota(jnp.int32, 16) * 16
        for g in range(16):
            facc = zeros
            for s in range(16):
                facc = facc + plsc.load_gather(fine_v, [lanes + (256 * g + s)])
            fold_v[pl.ds(g * 16, 16)] = facc

        pltpu.sync_copy(fold_v, out_hbm.at[wid])

    return hist_kernel(x, tcmm, sc_los, sc_his)


def kernel(x, bins, min, max):
    half = N // 2
    tcmm = _minmax_tc(x.reshape(N // _MM_COLS, _MM_COLS), half // _MM_COLS)
    sc_los, sc_his = _minmax_sc(x, half)
    partials = _sc_hist(x, tcmm, sc_los, sc_his)
    return jnp.sum(partials, axis=0)


# speedup vs baseline: 1.2368x; 1.2368x over previous
import functools

import jax
import jax.numpy as jnp
from jax import lax
from jax.experimental import pallas as pl
from jax.experimental.pallas import tpu as pltpu
from jax.experimental.pallas import tpu_sc as plsc

N = 16777216
NBINS = 256
_FINE = NBINS * 16


_MM_ROWS = 256
_MM_COLS = 8192
_MM_ACC = (8, 1024)


def _minmax_body(x_ref, mm_ref, amin_ref, amax_ref):
    i = pl.program_id(0)
    blk = x_ref[...].reshape(-1, *_MM_ACC)
    bmin = jnp.min(blk, axis=0)
    bmax = jnp.max(blk, axis=0)

    @pl.when(i == 0)
    def _init():
        amin_ref[...] = bmin
        amax_ref[...] = bmax

    @pl.when(i > 0)
    def _acc():
        amin_ref[...] = jnp.minimum(amin_ref[...], bmin)
        amax_ref[...] = jnp.maximum(amax_ref[...], bmax)

    @pl.when(i == pl.num_programs(0) - 1)
    def _fin():
        mm_ref[...] = jnp.concatenate([
            jnp.full((1, 16), jnp.min(amin_ref[...]), jnp.float32),
            jnp.full((1, 16), jnp.max(amax_ref[...]), jnp.float32),
        ])


def _minmax_tc(x2d, cover_rows):
    grid = cover_rows // _MM_ROWS
    return pl.pallas_call(
        _minmax_body,
        grid=(grid,),
        in_specs=[pl.BlockSpec((_MM_ROWS, _MM_COLS), lambda i: (i, 0))],
        out_specs=pl.BlockSpec((2, 16), lambda i: (0, 0)),
        out_shape=jax.ShapeDtypeStruct((2, 16), jnp.float32),
        scratch_shapes=[
            pltpu.VMEM(_MM_ACC, jnp.float32),
            pltpu.VMEM(_MM_ACC, jnp.float32),
        ],
    )(x2d)


_MMSC_CHUNK = 32768
_MMSC_UNROLL = 8


def _minmax_sc(x, lo_elem):
    nh = N - lo_elem
    pw = nh // _NW
    nchunk = pw // _MMSC_CHUNK
    mesh = plsc.VectorSubcoreMesh(core_axis_name="c", subcore_axis_name="s")

    @functools.partial(
        pl.kernel,
        mesh=mesh,
        out_type=(
            jax.ShapeDtypeStruct((_NW, 16), jnp.float32),
            jax.ShapeDtypeStruct((_NW, 16), jnp.float32),
        ),
        scratch_types=[
            pltpu.VMEM((2, _MMSC_CHUNK), jnp.float32),
            pltpu.VMEM((16,), jnp.float32),
            pltpu.VMEM((16,), jnp.float32),
            pltpu.SemaphoreType.DMA,
            pltpu.SemaphoreType.DMA,
        ],
        compiler_params=pltpu.CompilerParams(needs_layout_passes=False),
    )
    def mm_kernel(x_hbm, lo_hbm, hi_hbm, bufs_v, lo_v, hi_v, sem0, sem1):
        cid = lax.axis_index("c")
        sid = lax.axis_index("s")
        wid = sid * 2 + cid
        base = lo_elem + wid * pw
        sems = (sem0, sem1)

        def start(c):
            b = c & 1
            return pltpu.async_copy(
                x_hbm.at[pl.ds(base + c * _MMSC_CHUNK, _MMSC_CHUNK)],
                bufs_v.at[b], sems[b],
            )

        def make_vec_body(b):
            def vec_body(j, carry):
                off = j * (16 * _MMSC_UNROLL)
                mins = list(carry[:_MMSC_UNROLL])
                maxs = list(carry[_MMSC_UNROLL:])
                for k in range(_MMSC_UNROLL):
                    v = bufs_v[b, pl.ds(off + k * 16, 16)]
                    mins[k] = jnp.minimum(mins[k], v)
                    maxs[k] = jnp.maximum(maxs[k], v)
                return tuple(mins) + tuple(maxs)
            return vec_body

        pinf = jnp.full((16,), jnp.inf, dtype=jnp.float32)
        ninf = jnp.full((16,), -jnp.inf, dtype=jnp.float32)
        carry = (pinf,) * _MMSC_UNROLL + (ninf,) * _MMSC_UNROLL

        copies = [start(0)]
        for c in range(nchunk):
            if c + 1 < nchunk:
                copies.append(start(c + 1))
            copies[c].wait()
            carry = lax.fori_loop(
                0, _MMSC_CHUNK // (16 * _MMSC_UNROLL), make_vec_body(c & 1),
                carry)

        amin, amax = carry[0], carry[_MMSC_UNROLL]
        for k in range(1, _MMSC_UNROLL):
            amin = jnp.minimum(amin, carry[k])
            amax = jnp.maximum(amax, carry[_MMSC_UNROLL + k])
        lo_v[...] = amin
        hi_v[...] = amax
        pltpu.sync_copy(lo_v, lo_hbm.at[wid])
        pltpu.sync_copy(hi_v, hi_hbm.at[wid])

    return mm_kernel(x)


_NW = 32
_PW = N // _NW
_CHUNK = 32768
_NCHUNK = _PW // _CHUNK
_UNROLL = 32


def _sc_hist(x, tcmm, sc_los, sc_his):
    mesh = plsc.VectorSubcoreMesh(core_axis_name="c", subcore_axis_name="s")

    @functools.partial(
        pl.kernel,
        mesh=mesh,
        out_type=jax.ShapeDtypeStruct((_NW, _FINE), jnp.float32),
        scratch_types=[
            pltpu.VMEM((2, 16), jnp.float32),
            pltpu.VMEM((_NW, 16), jnp.float32),
            pltpu.VMEM((_NW, 16), jnp.float32),
            pltpu.VMEM((2, _CHUNK), jnp.float32),
            pltpu.VMEM((_FINE,), jnp.float32),
            pltpu.SemaphoreType.DMA,
            pltpu.SemaphoreType.DMA,
        ],
        compiler_params=pltpu.CompilerParams(needs_layout_passes=False),
    )
    def hist_kernel(x_hbm, tcmm_hbm, los_hbm, his_hbm, out_hbm, tcmm_v,
                    los_v, his_v, bufs_v, fine_v, sem0, sem1):
        cid = lax.axis_index("c")
        sid = lax.axis_index("s")
        wid = sid * 2 + cid
        base = wid * _PW
        sems = (sem0, sem1)

        pltpu.sync_copy(tcmm_hbm, tcmm_v)
        pltpu.sync_copy(los_hbm, los_v)
        pltpu.sync_copy(his_hbm, his_v)
        lo_v = tcmm_v[0]
        hi_v = tcmm_v[1]
        for w in range(_NW):
            lo_v = jnp.minimum(lo_v, los_v[w])
            hi_v = jnp.maximum(hi_v, his_v[w])
        lo_v = jnp.broadcast_to(jnp.min(lo_v), (16,))
        hi_v = jnp.broadcast_to(jnp.max(hi_v), (16,))
        scale16_v = jnp.full((16,), 4096.0 * (1.0 - 2.0**-18),
                             dtype=jnp.float32) / (hi_v - lo_v)
        ones = jnp.full((16,), 1.0, dtype=jnp.float32)
        zeros = jnp.zeros((16,), dtype=jnp.float32)

        def zero_body(j, _):
            fine_v[pl.ds(j * 16, 16)] = zeros
            return 0

        lax.fori_loop(0, _FINE // 16, zero_body, 0)

        def start(c):
            b = c & 1
            return pltpu.async_copy(
                x_hbm.at[pl.ds(base + c * _CHUNK, _CHUNK)],
                bufs_v.at[b], sems[b],
            )

        def make_vec_body(b):
            def vec_body(j, acc):
                off = j * (16 * _UNROLL)
                idxs = []
                for k in range(_UNROLL):
                    v = bufs_v[b, pl.ds(off + k * 16, 16)]
                    idxs.append(((v - lo_v) * scale16_v).astype(jnp.int32))
                for idx in idxs:
                    plsc.addupdate_scatter(fine_v, [idx], ones)
                return acc
            return vec_body

        acc = jnp.zeros((16,), dtype=jnp.int32)
        copies = [start(0)]
        for c in range(_NCHUNK):
            if c + 1 < _NCHUNK:
                copies.append(start(c + 1))
            copies[c].wait()
            acc = lax.fori_loop(0, _CHUNK // (16 * _UNROLL),
                                make_vec_body(c & 1), acc)

        pltpu.sync_copy(fine_v, out_hbm.at[wid])

    return hist_kernel(x, tcmm, sc_los, sc_his)


def kernel(x, bins, min, max):
    half = N // 2
    tcmm = jnp.concatenate([jnp.full((1, 16), jnp.inf, jnp.float32),
                            jnp.full((1, 16), -jnp.inf, jnp.float32)])
    sc_los, sc_his = _minmax_sc(x, 0)
    partials = _sc_hist(x, tcmm, sc_los, sc_his)
    return jnp.sum(partials, axis=0).reshape(NBINS, 16).sum(axis=1)
